# Initial kernel scaffold; baseline (speedup 1.0000x reference)
import jax
import jax.numpy as jnp
from jax.experimental import pallas as pl


def kernel(x, edge_index, W_l, W_r, b_l):
    raise NotImplementedError("write your pallas kernel here")


# speedup vs baseline: 4.0046x; 4.0046x over previous
import jax
import jax.numpy as jnp
from jax import lax
from jax.experimental import pallas as pl
from jax.experimental.pallas import tpu as pltpu
from jax.experimental.pallas import tpu_sc as plsc

N_NODES = 10000
D = 128
N_EDGES = 320000

NC = 2
NS = 16
NW = NC * NS

CHUNK = 128
K = 80
KB = K // 8
E_PAD = NW * K * CHUNK
N_PAD = 10112
ROWS_PER_TILE = N_PAD // NS


def _fill_rows(rows_v, val):
    @pl.loop(0, CHUNK)
    def _fill(i):
        v16 = jnp.zeros((16,), jnp.float32) + val
        for col in range(D // 16):
            rows_v[i, pl.ds(col * 16, 16)] = v16


def _zero_acc(rows_v, acc_s, base):
    for k in range(ROWS_PER_TILE // CHUNK):
        pltpu.sync_copy(rows_v, acc_s.at[pl.ds(base + k * CHUNK, CHUNK)])
    rem = ROWS_PER_TILE % CHUNK
    if rem:
        off = base + (ROWS_PER_TILE // CHUNK) * CHUNK
        pltpu.sync_copy(rows_v.at[pl.ds(0, rem)], acc_s.at[pl.ds(off, rem)])


def _copy_out(acc_s, out, c, base):
    for k in range(ROWS_PER_TILE // CHUNK):
        pltpu.sync_copy(acc_s.at[pl.ds(base + k * CHUNK, CHUNK)],
                        out.at[c, pl.ds(base + k * CHUNK, CHUNK)])
    rem = ROWS_PER_TILE % CHUNK
    if rem:
        off = base + (ROWS_PER_TILE // CHUNK) * CHUNK
        pltpu.sync_copy(acc_s.at[pl.ds(off, rem)],
                        out.at[c, pl.ds(off, rem)])


def _sc_body(x_hbm, idx_hbm, acc_out, deg_out, idx_b, rows_v, acc_s, gsem):
    c = lax.axis_index("c")
    s = lax.axis_index("s")
    wid = c * NS + s
    base = s * ROWS_PER_TILE

    _fill_rows(rows_v, 0.0)
    _zero_acc(rows_v, acc_s, base)
    plsc.subcore_barrier()

    @pl.loop(0, KB)
    def _edges(jb):
        pltpu.sync_copy(idx_hbm.at[wid, jb], idx_b)
        for b in range(8):
            pltpu.async_copy(x_hbm.at[idx_b.at[0, b]], rows_v, gsem).wait()
            pltpu.sync_copy(rows_v, acc_s.at[idx_b.at[1, b]], add=True)

    plsc.subcore_barrier()
    _copy_out(acc_s, acc_out, c, base)
    plsc.subcore_barrier()

    _fill_rows(rows_v, 0.0)
    _zero_acc(rows_v, acc_s, base)
    _fill_rows(rows_v, 1.0)
    plsc.subcore_barrier()

    @pl.loop(0, KB)
    def _deg(jb):
        pltpu.sync_copy(idx_hbm.at[wid, jb], idx_b)
        for b in range(8):
            pltpu.sync_copy(rows_v, acc_s.at[idx_b.at[1, b]], add=True)

    plsc.subcore_barrier()
    _copy_out(acc_s, deg_out, c, base)


def _sc_aggregate(x, idx_r):
    mesh = plsc.VectorSubcoreMesh(core_axis_name="c", subcore_axis_name="s")
    f = pl.kernel(
        _sc_body,
        out_type=(
            jax.ShapeDtypeStruct((NC, N_PAD, D), jnp.float32),
            jax.ShapeDtypeStruct((NC, N_PAD, D), jnp.float32),
        ),
        mesh=mesh,
        scratch_types=[
            pltpu.VMEM((2, 8, CHUNK), jnp.int32),
            pltpu.VMEM((CHUNK, D), jnp.float32),
            pltpu.VMEM_SHARED((N_PAD, D), jnp.float32),
            pltpu.SemaphoreType.DMA,
        ],
    )
    return f(x, idx_r)


def _tc_body(acc_ref, deg_ref, x_ref, wl_ref, wr_ref, b_ref, out_ref):
    acc = acc_ref[0] + acc_ref[1]
    deg = deg_ref[0, :, 0:1] + deg_ref[1, :, 0:1]
    mean = acc / jnp.maximum(deg, 1.0)
    dn = (((1,), (1,)), ((), ()))
    out = lax.dot_general(mean, wl_ref[...], dn,
                          preferred_element_type=jnp.float32)
    out = out + lax.dot_general(x_ref[...], wr_ref[...], dn,
                                preferred_element_type=jnp.float32)
    out_ref[...] = jnp.maximum(out + b_ref[...], 0.0)


def _tc_combine(acc_p, deg_p, x, W_l, W_r, b_l):
    B = 1000
    grid = (N_NODES // B,)
    return pl.pallas_call(
        _tc_body,
        grid=grid,
        in_specs=[
            pl.BlockSpec((NC, B, D), lambda i: (0, i, 0)),
            pl.BlockSpec((NC, B, D), lambda i: (0, i, 0)),
            pl.BlockSpec((B, D), lambda i: (i, 0)),
            pl.BlockSpec((D, D), lambda i: (0, 0)),
            pl.BlockSpec((D, D), lambda i: (0, 0)),
            pl.BlockSpec((1, D), lambda i: (0, 0)),
        ],
        out_specs=pl.BlockSpec((B, D), lambda i: (i, 0)),
        out_shape=jax.ShapeDtypeStruct((N_NODES, D), jnp.float32),
    )(acc_p, deg_p, x, W_l, W_r, b_l)


@jax.jit
def kernel(x, edge_index, W_l, W_r, b_l):
    src = edge_index[0].astype(jnp.int32)
    dst = edge_index[1].astype(jnp.int32)
    pad = E_PAD - N_EDGES
    src_r = jnp.concatenate(
        [src, jnp.zeros((pad,), jnp.int32)]).reshape(NW, KB, 1, 8, CHUNK)
    dst_r = jnp.concatenate(
        [dst, jnp.full((pad,), N_NODES, jnp.int32)]).reshape(NW, KB, 1, 8, CHUNK)
    idx_r = jnp.concatenate([src_r, dst_r], axis=2)
    acc_p, deg_p = _sc_aggregate(x, idx_r)
    return _tc_combine(acc_p, deg_p, x, W_l, W_r, b_l.reshape(1, D))


# speedup vs baseline: 4.4134x; 1.1021x over previous
import jax
import jax.numpy as jnp
from jax import lax
from jax.experimental import pallas as pl
from jax.experimental.pallas import tpu as pltpu
from jax.experimental.pallas import tpu_sc as plsc

N_NODES = 10000
D = 128
N_EDGES = 320000

NC = 2
NS = 16
NW = NC * NS

CHUNK = 128
K = 80
KB = K // 8
E_PAD = NW * K * CHUNK
N_PAD = 10112
ROWS_PER_TILE = N_PAD // NS


def _fill_rows(rows_v, val):
    @pl.loop(0, CHUNK)
    def _fill(i):
        v16 = jnp.zeros((16,), jnp.float32) + val
        for col in range(D // 16):
            rows_v[i, pl.ds(col * 16, 16)] = v16


def _zero_acc(rows_v, acc_s, base):
    for k in range(ROWS_PER_TILE // CHUNK):
        pltpu.sync_copy(rows_v, acc_s.at[pl.ds(base + k * CHUNK, CHUNK)])
    rem = ROWS_PER_TILE % CHUNK
    if rem:
        off = base + (ROWS_PER_TILE // CHUNK) * CHUNK
        pltpu.sync_copy(rows_v.at[pl.ds(0, rem)], acc_s.at[pl.ds(off, rem)])


def _copy_out(acc_s, out, c, base):
    for k in range(ROWS_PER_TILE // CHUNK):
        pltpu.sync_copy(acc_s.at[pl.ds(base + k * CHUNK, CHUNK)],
                        out.at[c, pl.ds(base + k * CHUNK, CHUNK)])
    rem = ROWS_PER_TILE % CHUNK
    if rem:
        off = base + (ROWS_PER_TILE // CHUNK) * CHUNK
        pltpu.sync_copy(acc_s.at[pl.ds(off, rem)],
                        out.at[c, pl.ds(off, rem)])


def _sc_body(x_hbm, idx_hbm, acc_out, deg_out, idx_b, rows_v, rows_w,
             acc_s, gsem, gsem2):
    c = lax.axis_index("c")
    s = lax.axis_index("s")
    wid = c * NS + s
    base = s * ROWS_PER_TILE

    _fill_rows(rows_v, 0.0)
    _zero_acc(rows_v, acc_s, base)
    plsc.subcore_barrier()

    @pl.loop(0, KB)
    def _edges(jb):
        pltpu.sync_copy(idx_hbm.at[wid, jb], idx_b)
        bufs = (rows_v, rows_w)
        sems = (gsem, gsem2)
        cps = [None] * 8
        cps[0] = pltpu.async_copy(x_hbm.at[idx_b.at[0, 0]], bufs[0], sems[0])
        for b in range(8):
            if b < 7:
                cps[b + 1] = pltpu.async_copy(
                    x_hbm.at[idx_b.at[0, b + 1]],
                    bufs[(b + 1) % 2], sems[(b + 1) % 2])
            cps[b].wait()
            pltpu.sync_copy(bufs[b % 2], acc_s.at[idx_b.at[1, b]], add=True)

    plsc.subcore_barrier()
    _copy_out(acc_s, acc_out, c, base)
    plsc.subcore_barrier()

    _fill_rows(rows_v, 0.0)
    _zero_acc(rows_v, acc_s, base)
    _fill_rows(rows_v, 1.0)
    plsc.subcore_barrier()

    @pl.loop(0, KB)
    def _deg(jb):
        pltpu.sync_copy(idx_hbm.at[wid, jb], idx_b)
        for b in range(8):
            pltpu.sync_copy(rows_v, acc_s.at[idx_b.at[1, b]], add=True)

    plsc.subcore_barrier()
    _copy_out(acc_s, deg_out, c, base)


def _sc_aggregate(x, idx_r):
    mesh = plsc.VectorSubcoreMesh(core_axis_name="c", subcore_axis_name="s")
    f = pl.kernel(
        _sc_body,
        out_type=(
            jax.ShapeDtypeStruct((NC, N_PAD, D), jnp.float32),
            jax.ShapeDtypeStruct((NC, N_PAD, D), jnp.float32),
        ),
        mesh=mesh,
        scratch_types=[
            pltpu.VMEM((2, 8, CHUNK), jnp.int32),
            pltpu.VMEM((CHUNK, D), jnp.float32),
            pltpu.VMEM((CHUNK, D), jnp.float32),
            pltpu.VMEM_SHARED((N_PAD, D), jnp.float32),
            pltpu.SemaphoreType.DMA,
            pltpu.SemaphoreType.DMA,
        ],
    )
    return f(x, idx_r)


def _tc_body(acc_ref, deg_ref, x_ref, wl_ref, wr_ref, b_ref, out_ref):
    acc = acc_ref[0] + acc_ref[1]
    deg = deg_ref[0, :, 0:1] + deg_ref[1, :, 0:1]
    mean = acc / jnp.maximum(deg, 1.0)
    dn = (((1,), (1,)), ((), ()))
    out = lax.dot_general(mean, wl_ref[...], dn,
                          preferred_element_type=jnp.float32)
    out = out + lax.dot_general(x_ref[...], wr_ref[...], dn,
                                preferred_element_type=jnp.float32)
    out_ref[...] = jnp.maximum(out + b_ref[...], 0.0)


def _tc_combine(acc_p, deg_p, x, W_l, W_r, b_l):
    B = 1000
    grid = (N_NODES // B,)
    return pl.pallas_call(
        _tc_body,
        grid=grid,
        in_specs=[
            pl.BlockSpec((NC, B, D), lambda i: (0, i, 0)),
            pl.BlockSpec((NC, B, D), lambda i: (0, i, 0)),
            pl.BlockSpec((B, D), lambda i: (i, 0)),
            pl.BlockSpec((D, D), lambda i: (0, 0)),
            pl.BlockSpec((D, D), lambda i: (0, 0)),
            pl.BlockSpec((1, D), lambda i: (0, 0)),
        ],
        out_specs=pl.BlockSpec((B, D), lambda i: (i, 0)),
        out_shape=jax.ShapeDtypeStruct((N_NODES, D), jnp.float32),
    )(acc_p, deg_p, x, W_l, W_r, b_l)


@jax.jit
def kernel(x, edge_index, W_l, W_r, b_l):
    src = edge_index[0].astype(jnp.int32)
    dst = edge_index[1].astype(jnp.int32)
    pad = E_PAD - N_EDGES
    src_r = jnp.concatenate(
        [src, jnp.zeros((pad,), jnp.int32)]).reshape(NW, KB, 1, 8, CHUNK)
    dst_r = jnp.concatenate(
        [dst, jnp.full((pad,), N_NODES, jnp.int32)]).reshape(NW, KB, 1, 8, CHUNK)
    idx_r = jnp.concatenate([src_r, dst_r], axis=2)
    acc_p, deg_p = _sc_aggregate(x, idx_r)
    return _tc_combine(acc_p, deg_p, x, W_l, W_r, b_l.reshape(1, D))
